# baseline (speedup 1.0000x reference)
import functools

import jax
import jax.numpy as jnp
from jax import lax
from jax.experimental import pallas as pl
from jax.experimental.pallas import tpu as pltpu
from jax.experimental.pallas import tpu_sc as plsc

B = 16384
D = 16
NC = 2
NS = 16
NW = NC * NS
BPW = B // NW
CH = 128
NCH = BPW // CH
L = 16
NG = BPW // L


def _sc_body(bu_hbm, bp_hbm, bn_hbm, ut_hbm, it_hbm,
             outp_hbm, outn_hbm,
             idx_u, idx_p, idx_n, u_rows, i_rows, j_rows,
             accp, accn, sem):
  wid = lax.axis_index("s") * NC + lax.axis_index("c")

  pltpu.sync_copy(bu_hbm.at[wid], idx_u)
  pltpu.sync_copy(bp_hbm.at[wid], idx_p)
  pltpu.sync_copy(bn_hbm.at[wid], idx_n)

  copies = []
  for c in range(NCH):
    dst = pl.ds(c * CH, CH)
    copies.append(pltpu.make_async_copy(ut_hbm.at[idx_u.at[c]],
                                        u_rows.at[dst], sem))
    copies.append(pltpu.make_async_copy(it_hbm.at[idx_p.at[c]],
                                        i_rows.at[dst], sem))
    copies.append(pltpu.make_async_copy(it_hbm.at[idx_n.at[c]],
                                        j_rows.at[dst], sem))
  for cp in copies:
    cp.start()
  for cp in copies:
    cp.wait()

  lane = lax.iota(jnp.int32, L)

  def group(g, carry):
    base = pl.multiple_of(g * L, L)
    rows = base + lane
    ap = jnp.zeros((L,), jnp.float32)
    an = jnp.zeros((L,), jnp.float32)
    for d in range(D):
      cols = jnp.full((L,), d, jnp.int32)
      uu = plsc.load_gather(u_rows, [rows, cols])
      ii = plsc.load_gather(i_rows, [rows, cols])
      jj = plsc.load_gather(j_rows, [rows, cols])
      ap = ap + uu * ii
      an = an + uu * jj
    accp[pl.ds(base, L)] = ap
    accn[pl.ds(base, L)] = an
    return carry

  lax.fori_loop(0, NG, group, 0)

  out = pl.ds(wid * BPW, BPW)
  pltpu.sync_copy(accp, outp_hbm.at[out])
  pltpu.sync_copy(accn, outn_hbm.at[out])


@jax.jit
def kernel(batch_user, batch_pos_item, batch_neg_item, user_table, item_table):
  bu = batch_user.reshape(NW, NCH, CH)
  bp = batch_pos_item.reshape(NW, NCH, CH)
  bn = batch_neg_item.reshape(NW, NCH, CH)

  mesh = plsc.VectorSubcoreMesh(core_axis_name="c", subcore_axis_name="s",
                                num_cores=NC, num_subcores=NS)
  run = pl.kernel(
      _sc_body,
      out_type=(jax.ShapeDtypeStruct((B,), jnp.float32),
                jax.ShapeDtypeStruct((B,), jnp.float32)),
      mesh=mesh,
      scratch_types=[
          pltpu.VMEM((NCH, CH), jnp.int32),
          pltpu.VMEM((NCH, CH), jnp.int32),
          pltpu.VMEM((NCH, CH), jnp.int32),
          pltpu.VMEM((BPW, D), jnp.float32),
          pltpu.VMEM((BPW, D), jnp.float32),
          pltpu.VMEM((BPW, D), jnp.float32),
          pltpu.VMEM((BPW,), jnp.float32),
          pltpu.VMEM((BPW,), jnp.float32),
          pltpu.SemaphoreType.DMA,
      ],
      compiler_params=pltpu.CompilerParams(needs_layout_passes=False,
                                           use_tc_tiling_on_sc=False),
  )
  pos, neg = run(bu, bp, bn, user_table, item_table)
  return (pos.reshape(B, 1), neg.reshape(B, 1))
